# Initial kernel scaffold; baseline (speedup 1.0000x reference)
import jax
import jax.numpy as jnp
from jax.experimental import pallas as pl


def kernel(X, Seed, T, D, C, edge_index, W_node, W_seed, W_time, W_deg, W_clu, Wq, Wk, Wv, Wo, W1, b1, W2, b2):
    raise NotImplementedError("write your pallas kernel here")


# speedup vs baseline: 49.7777x; 49.7777x over previous
import functools

import jax
import jax.numpy as jnp
from jax import lax
from jax.experimental import pallas as pl
from jax.experimental.pallas import tpu as pltpu
from jax.experimental.pallas import tpu_sc as plsc

N = 10000
E = 320000
D_IN = 128
HEADS = 8
DH = 16

NC = 2
NS = 16
NW = NC * NS
EPW = E // NW
CH = 80
NCHUNK = EPW // CH
RPT = N // NS


def _qkv_body(xc_ref, wc_ref, wq_ref, wk_ref, wv_ref,
              h_ref, q_ref, k_ref, v_ref):
    h = jnp.dot(xc_ref[...], wc_ref[...], preferred_element_type=jnp.float32)
    h_ref[...] = h
    q_ref[...] = jnp.dot(h, wq_ref[...], preferred_element_type=jnp.float32)
    k_ref[...] = jnp.dot(h, wk_ref[...], preferred_element_type=jnp.float32)
    v_ref[...] = jnp.dot(h, wv_ref[...], preferred_element_type=jnp.float32)


_qkv_call = pl.pallas_call(
    _qkv_body,
    out_shape=[jax.ShapeDtypeStruct((N, D_IN), jnp.float32)] * 4,
)


def _edge_body(q_hbm, k_hbm, v_hbm, src_hbm, dst_hbm, agg_out, den_out,
               src_v, dst_v, qr, kr, vr, er, agg_sh, den_sh,
               sem_q, sem_k, sem_v):
    cid = lax.axis_index("c")
    sid = lax.axis_index("s")
    lane = lax.iota(jnp.int32, 16)
    zvec = jnp.zeros((16,), jnp.float32)

    def zrow(r, carry):
        for j in range(D_IN // 16):
            vr[r, pl.ds(j * 16, 16)] = zvec
        er[r, :] = zvec
        return carry

    lax.fori_loop(0, CH, zrow, 0)
    rb = sid * RPT
    for i in range(7):
        pltpu.sync_copy(vr, agg_sh.at[pl.ds(rb + i * CH, CH)])
        pltpu.sync_copy(er, den_sh.at[pl.ds(rb + i * CH, CH)])
    pltpu.sync_copy(vr.at[pl.ds(0, RPT - 7 * CH)],
                    agg_sh.at[pl.ds(rb + 7 * CH, RPT - 7 * CH)])
    pltpu.sync_copy(er.at[pl.ds(0, RPT - 7 * CH)],
                    den_sh.at[pl.ds(rb + 7 * CH, RPT - 7 * CH)])
    plsc.subcore_barrier()

    base = (cid * NS + sid) * EPW

    def chunk(i, carry):
        eb = base + i * CH
        pltpu.sync_copy(src_hbm.at[pl.ds(eb, CH)], src_v)
        pltpu.sync_copy(dst_hbm.at[pl.ds(eb, CH)], dst_v)
        cq = pltpu.async_copy(q_hbm.at[dst_v], qr, sem_q)
        ck = pltpu.async_copy(k_hbm.at[src_v], kr, sem_k)
        cv = pltpu.async_copy(v_hbm.at[src_v], vr, sem_v)
        cq.wait()
        ck.wait()
        cv.wait()

        def edge(e, c2):
            sv = zvec
            for hh in range(HEADS):
                qv = qr[e, pl.ds(hh * DH, DH)]
                kv = kr[e, pl.ds(hh * DH, DH)]
                s = jnp.sum(qv * kv)
                sv = jnp.where(lane == hh, s, sv)
            ex = jnp.exp(sv * 0.25)
            ex = jnp.where(lane < HEADS, ex, 0.0)
            er[e, :] = ex
            for hh in range(HEADS):
                w = ex[hh]
                vr[e, pl.ds(hh * DH, DH)] = vr[e, pl.ds(hh * DH, DH)] * w
            return c2

        lax.fori_loop(0, CH, edge, 0)
        pltpu.sync_copy(vr, agg_sh.at[dst_v], add=True)
        pltpu.sync_copy(er, den_sh.at[dst_v], add=True)
        return carry

    lax.fori_loop(0, NCHUNK, chunk, 0)
    plsc.subcore_barrier()

    pltpu.sync_copy(agg_sh.at[pl.ds(rb, RPT)], agg_out.at[cid, pl.ds(rb, RPT)])
    pltpu.sync_copy(den_sh.at[pl.ds(rb, RPT)], den_out.at[cid, pl.ds(rb, RPT)])


_edge_call = functools.partial(
    pl.kernel,
    out_type=[jax.ShapeDtypeStruct((NC, N, D_IN), jnp.float32),
              jax.ShapeDtypeStruct((NC, N, 16), jnp.float32)],
    mesh=plsc.VectorSubcoreMesh(core_axis_name="c", subcore_axis_name="s"),
    compiler_params=pltpu.CompilerParams(use_tc_tiling_on_sc=False,
                                         needs_layout_passes=False),
    scratch_types=[
        pltpu.VMEM((CH,), jnp.int32),
        pltpu.VMEM((CH,), jnp.int32),
        pltpu.VMEM((CH, D_IN), jnp.float32),
        pltpu.VMEM((CH, D_IN), jnp.float32),
        pltpu.VMEM((CH, D_IN), jnp.float32),
        pltpu.VMEM((CH, 16), jnp.float32),
        pltpu.VMEM_SHARED((N, D_IN), jnp.float32),
        pltpu.VMEM_SHARED((N, 16), jnp.float32),
        pltpu.SemaphoreType.DMA,
        pltpu.SemaphoreType.DMA,
        pltpu.SemaphoreType.DMA,
    ],
)(_edge_body)


def _head_body(agg0_ref, den0_ref, agg1_ref, den1_ref, h_ref,
               wo_ref, w1_ref, b1_ref, w2_ref, b2_ref, bmat_ref, out_ref):
    aggs = agg0_ref[...] + agg1_ref[...]
    dens = den0_ref[...] + den1_ref[...]
    rec = 1.0 / (dens + 1e-16)
    rec128 = jnp.dot(rec, bmat_ref[...], preferred_element_type=jnp.float32)
    attn = aggs * rec128
    out = jnp.dot(attn, wo_ref[...], preferred_element_type=jnp.float32)
    out = out + h_ref[...]
    hm = jnp.maximum(
        jnp.dot(out, w1_ref[...], preferred_element_type=jnp.float32)
        + b1_ref[...], 0.0)
    out_ref[...] = (jnp.dot(hm, w2_ref[...], preferred_element_type=jnp.float32)
                    + b2_ref[...])


_head_call = pl.pallas_call(
    _head_body,
    out_shape=jax.ShapeDtypeStruct((N, 1), jnp.float32),
)


def kernel(X, Seed, T, D, C, edge_index,
           W_node, W_seed, W_time, W_deg, W_clu,
           Wq, Wk, Wv, Wo, W1, b1, W2, b2):
    zc = jnp.zeros((N, 3), jnp.float32)
    xc = jnp.concatenate([X, Seed, T, D, C, zc], axis=1)
    wc = jnp.concatenate(
        [W_node, W_seed, W_time, W_deg, W_clu,
         jnp.zeros((3, D_IN), jnp.float32)], axis=0)

    h, q, k, v = _qkv_call(xc, wc, Wq, Wk, Wv)

    src = edge_index[0]
    dst = edge_index[1]
    agg, den = _edge_call(q, k, v, src, dst)

    eye = jnp.eye(16, dtype=jnp.float32)[:, :HEADS]
    bmat = jnp.repeat(eye, DH, axis=1)

    predX = _head_call(agg[0], den[0], agg[1], den[1], h,
                       Wo, W1, b1.reshape(1, D_IN), W2,
                       b2.reshape(1, 1), bmat)
    return predX


# speedup vs baseline: 79.9535x; 1.6062x over previous
import functools

import jax
import jax.numpy as jnp
from jax import lax
from jax.experimental import pallas as pl
from jax.experimental.pallas import tpu as pltpu
from jax.experimental.pallas import tpu_sc as plsc

N = 10000
E = 320000
D_IN = 128
HEADS = 8
DH = 16

NC = 2
NS = 16
NW = NC * NS
EPW = E // NW
CH = 40
NCHUNK = EPW // CH
RPT = N // NS


def _qkv_body(xc_ref, wc_ref, wq_ref, wk_ref, wv_ref,
              h_ref, q_ref, k_ref, v_ref):
    h = jnp.dot(xc_ref[...], wc_ref[...], preferred_element_type=jnp.float32)
    h_ref[...] = h
    q_ref[...] = jnp.dot(h, wq_ref[...],
                         preferred_element_type=jnp.float32) * 0.25
    k_ref[...] = jnp.dot(h, wk_ref[...], preferred_element_type=jnp.float32)
    v_ref[...] = jnp.dot(h, wv_ref[...], preferred_element_type=jnp.float32)


_qkv_call = pl.pallas_call(
    _qkv_body,
    out_shape=[jax.ShapeDtypeStruct((N, D_IN), jnp.float32)] * 4,
)


def _edge_body(q_hbm, k_hbm, v_hbm, src_hbm, dst_hbm, agg_out, den_out,
               src_v0, dst_v0, qr0, kr0, vr0, er0,
               src_v1, dst_v1, qr1, kr1, vr1, er1,
               agg_sh, den_sh,
               sem_q0, sem_k0, sem_v0, sem_q1, sem_k1, sem_v1):
    cid = lax.axis_index("c")
    sid = lax.axis_index("s")
    lane = lax.iota(jnp.int32, 16)
    zvec = jnp.zeros((16,), jnp.float32)
    bufs = ((src_v0, dst_v0, qr0, kr0, vr0, er0, sem_q0, sem_k0, sem_v0),
            (src_v1, dst_v1, qr1, kr1, vr1, er1, sem_q1, sem_k1, sem_v1))

    def zrow(r, carry):
        for j in range(D_IN // 16):
            vr0[r, pl.ds(j * 16, 16)] = zvec
        er0[r, :] = zvec
        return carry

    lax.fori_loop(0, CH, zrow, 0)
    rb = sid * RPT
    nz = RPT // CH
    rem = RPT - nz * CH

    def zcopy(i, carry):
        pltpu.sync_copy(vr0, agg_sh.at[pl.ds(rb + i * CH, CH)])
        pltpu.sync_copy(er0, den_sh.at[pl.ds(rb + i * CH, CH)])
        return carry

    lax.fori_loop(0, nz, zcopy, 0)
    pltpu.sync_copy(vr0.at[pl.ds(0, rem)],
                    agg_sh.at[pl.ds(rb + nz * CH, rem)])
    pltpu.sync_copy(er0.at[pl.ds(0, rem)],
                    den_sh.at[pl.ds(rb + nz * CH, rem)])
    plsc.subcore_barrier()

    base = (cid * NS + sid) * EPW

    def start(g, b):
        src_v, dst_v, qr, kr, vr, er, sem_q, sem_k, sem_v = bufs[b]
        eb = base + g * CH
        pltpu.sync_copy(src_hbm.at[pl.ds(eb, CH)], src_v)
        pltpu.sync_copy(dst_hbm.at[pl.ds(eb, CH)], dst_v)
        pltpu.async_copy(q_hbm.at[dst_v], qr, sem_q)
        pltpu.async_copy(k_hbm.at[src_v], kr, sem_k)
        pltpu.async_copy(v_hbm.at[src_v], vr, sem_v)

    def compute(b):
        src_v, dst_v, qr, kr, vr, er, sem_q, sem_k, sem_v = bufs[b]
        pltpu.make_async_copy(q_hbm.at[dst_v], qr, sem_q).wait()
        pltpu.make_async_copy(k_hbm.at[src_v], kr, sem_k).wait()
        pltpu.make_async_copy(v_hbm.at[src_v], vr, sem_v).wait()

        @plsc.parallel_loop(0, CH, unroll=4)
        def edge(e):
            terms = []
            for hh in range(HEADS):
                qv = qr[e, pl.ds(hh * DH, DH)]
                kv = kr[e, pl.ds(hh * DH, DH)]
                s = jnp.sum(qv * kv)
                terms.append(jnp.where(lane == hh, s, 0.0))
            sv = (((terms[0] + terms[1]) + (terms[2] + terms[3]))
                  + ((terms[4] + terms[5]) + (terms[6] + terms[7])))
            ex = jnp.exp(sv)
            ex = jnp.where(lane < HEADS, ex, 0.0)
            er[e, :] = ex
            for hh in range(HEADS):
                w = ex.at[jnp.full((16,), hh, jnp.int32)].get(
                    mode="promise_in_bounds")
                vr[e, pl.ds(hh * DH, DH)] = vr[e, pl.ds(hh * DH, DH)] * w

        pltpu.sync_copy(vr, agg_sh.at[dst_v], add=True)
        pltpu.sync_copy(er, den_sh.at[dst_v], add=True)

    start(0, 0)
    start(1, 1)

    def pair(i2, carry):
        g = i2 * 2
        compute(0)
        start(g + 2, 0)
        compute(1)
        start(g + 3, 1)
        return carry

    lax.fori_loop(0, NCHUNK // 2 - 1, pair, 0)
    compute(0)
    compute(1)
    plsc.subcore_barrier()

    pltpu.sync_copy(agg_sh.at[pl.ds(rb, RPT)], agg_out.at[cid, pl.ds(rb, RPT)])
    pltpu.sync_copy(den_sh.at[pl.ds(rb, RPT)], den_out.at[cid, pl.ds(rb, RPT)])


_edge_call = functools.partial(
    pl.kernel,
    out_type=[jax.ShapeDtypeStruct((NC, N, D_IN), jnp.float32),
              jax.ShapeDtypeStruct((NC, N, 16), jnp.float32)],
    mesh=plsc.VectorSubcoreMesh(core_axis_name="c", subcore_axis_name="s"),
    compiler_params=pltpu.CompilerParams(use_tc_tiling_on_sc=False,
                                         needs_layout_passes=False),
    scratch_types=(
        [pltpu.VMEM((CH,), jnp.int32),
         pltpu.VMEM((CH,), jnp.int32),
         pltpu.VMEM((CH, D_IN), jnp.float32),
         pltpu.VMEM((CH, D_IN), jnp.float32),
         pltpu.VMEM((CH, D_IN), jnp.float32),
         pltpu.VMEM((CH, 16), jnp.float32)] * 2
        + [pltpu.VMEM_SHARED((N, D_IN), jnp.float32),
           pltpu.VMEM_SHARED((N, 16), jnp.float32)]
        + [pltpu.SemaphoreType.DMA] * 6
    ),
)(_edge_body)


def _head_body(agg0_ref, den0_ref, agg1_ref, den1_ref, h_ref,
               wo_ref, w1_ref, b1_ref, w2_ref, b2_ref, bmat_ref, out_ref):
    aggs = agg0_ref[...] + agg1_ref[...]
    dens = den0_ref[...] + den1_ref[...]
    rec = 1.0 / (dens + 1e-16)
    rec128 = jnp.dot(rec, bmat_ref[...], preferred_element_type=jnp.float32)
    attn = aggs * rec128
    out = jnp.dot(attn, wo_ref[...], preferred_element_type=jnp.float32)
    out = out + h_ref[...]
    hm = jnp.maximum(
        jnp.dot(out, w1_ref[...], preferred_element_type=jnp.float32)
        + b1_ref[...], 0.0)
    out_ref[...] = (jnp.dot(hm, w2_ref[...], preferred_element_type=jnp.float32)
                    + b2_ref[...])


_head_call = pl.pallas_call(
    _head_body,
    out_shape=jax.ShapeDtypeStruct((N, 1), jnp.float32),
)


def kernel(X, Seed, T, D, C, edge_index,
           W_node, W_seed, W_time, W_deg, W_clu,
           Wq, Wk, Wv, Wo, W1, b1, W2, b2):
    zc = jnp.zeros((N, 3), jnp.float32)
    xc = jnp.concatenate([X, Seed, T, D, C, zc], axis=1)
    wc = jnp.concatenate(
        [W_node, W_seed, W_time, W_deg, W_clu,
         jnp.zeros((3, D_IN), jnp.float32)], axis=0)

    h, q, k, v = _qkv_call(xc, wc, Wq, Wk, Wv)

    src = edge_index[0]
    dst = edge_index[1]
    agg, den = _edge_call(q, k, v, src, dst)

    eye = jnp.eye(16, dtype=jnp.float32)[:, :HEADS]
    bmat = jnp.repeat(eye, DH, axis=1)

    predX = _head_call(agg[0], den[0], agg[1], den[1], h,
                       Wo, W1, b1.reshape(1, D_IN), W2,
                       b2.reshape(1, 1), bmat)
    return predX
